# Initial kernel scaffold; baseline (speedup 1.0000x reference)
import jax
import jax.numpy as jnp
from jax.experimental import pallas as pl


def kernel(x, emb_table, fc_w, fc_b):
    raise NotImplementedError("write your pallas kernel here")


# speedup vs baseline: 2.4562x; 2.4562x over previous
import functools

import jax
import jax.numpy as jnp
from jax import lax
from jax.experimental import pallas as pl
from jax.experimental.pallas import tpu as pltpu
from jax.experimental.pallas import tpu_sc as plsc

VOCAB = 1000000
EMBED = 32
BATCH = 4096
SEQ = 200

_INFO = plsc.get_sparse_core_info()
_NC = _INFO.num_cores
_NS = _INFO.num_subcores
_L = _INFO.num_lanes
_NW = _NC * _NS
_BPW = BATCH // _NW
_IPW = _BPW * SEQ
_CH0 = 128
_CH1 = SEQ - _CH0
_NBUF = 4
_PLEN = EMBED + 16


def _sc_body(x_hbm, params_hbm, table_hbm, out_hbm, *scratch):
    idx_v, params_v, outs_v = scratch[0], scratch[1], scratch[2]
    bufs_a = scratch[3:3 + _NBUF]
    bufs_b = scratch[3 + _NBUF:3 + 2 * _NBUF]
    sems = scratch[3 + 2 * _NBUF:3 + 3 * _NBUF]

    wid = lax.axis_index("s") * _NC + lax.axis_index("c")
    base = wid * _IPW

    pltpu.sync_copy(x_hbm.at[pl.ds(base, _IPW)], idx_v)
    pltpu.sync_copy(params_hbm, params_v)

    w0 = params_v[pl.ds(0, _L)]
    w1 = params_v[pl.ds(_L, _L)]
    bias_v = params_v[pl.ds(EMBED, _L)]

    def idx_off(e):
        return pl.multiple_of(e * SEQ, 8)

    def issue(e, slot):
        off = idx_off(e)
        pltpu.async_copy(
            table_hbm.at[idx_v.at[pl.ds(off, _CH0)]], bufs_a[slot], sems[slot])
        pltpu.async_copy(
            table_hbm.at[idx_v.at[pl.ds(off + _CH0, _CH1)]], bufs_b[slot],
            sems[slot])

    def wait(e, slot):
        off = idx_off(e)
        pltpu.make_async_copy(
            table_hbm.at[idx_v.at[pl.ds(off, _CH0)]], bufs_a[slot],
            sems[slot]).wait()
        pltpu.make_async_copy(
            table_hbm.at[idx_v.at[pl.ds(off + _CH0, _CH1)]], bufs_b[slot],
            sems[slot]).wait()

    def accum(buf, n, carry):
        def row(r, c):
            a0, a1 = c
            return a0 + buf[r, pl.ds(0, _L)], a1 + buf[r, pl.ds(_L, _L)]
        return lax.fori_loop(0, n, row, carry, unroll=8)

    for slot in range(_NBUF):
        issue(jnp.int32(slot), slot)

    lanes = lax.iota(jnp.int32, _L)
    gpl = _L // _NBUF

    def group(g, lanevec):
        for slot in range(_NBUF):
            e = g * _NBUF + slot
            wait(e, slot)
            zero = jnp.zeros((_L,), jnp.float32)
            a0, a1 = accum(bufs_a[slot], _CH0, (zero, zero))
            a0, a1 = accum(bufs_b[slot], _CH1, (a0, a1))
            s = jnp.sum(a0 * w0 + a1 * w1)
            lanevec = jnp.where(lanes == e % _L, s, lanevec)

            @pl.when(e + _NBUF < _BPW)
            def _():
                issue(e + _NBUF, slot)

        @pl.when(g % gpl == gpl - 1)
        def _():
            outs_v[pl.ds(pl.multiple_of((g // gpl) * _L, _L), _L)] = lanevec

        return lanevec

    lax.fori_loop(0, _BPW // _NBUF, group, jnp.zeros((_L,), jnp.float32))

    inv = jnp.float32(1.0 / SEQ)
    one = jnp.float32(1.0)
    for k in range(_BPW // _L):
        z = outs_v[pl.ds(k * _L, _L)] * inv + bias_v
        outs_v[pl.ds(k * _L, _L)] = one / (one + jnp.exp(-z))

    pltpu.sync_copy(outs_v, out_hbm.at[pl.ds(wid * _BPW, _BPW)])


@jax.jit
def _run(x_flat, params, emb_table):
    mesh = plsc.VectorSubcoreMesh(core_axis_name="c", subcore_axis_name="s")
    scratch = [
        pltpu.VMEM((_IPW,), jnp.int32),
        pltpu.VMEM((_PLEN,), jnp.float32),
        pltpu.VMEM((_BPW,), jnp.float32),
    ]
    scratch += [pltpu.VMEM((_CH0, EMBED), jnp.float32) for _ in range(_NBUF)]
    scratch += [pltpu.VMEM((_CH1, EMBED), jnp.float32) for _ in range(_NBUF)]
    scratch += [pltpu.SemaphoreType.DMA for _ in range(_NBUF)]
    return pl.kernel(
        _sc_body,
        jax.ShapeDtypeStruct((BATCH,), jnp.float32),
        mesh=mesh,
        scratch_types=scratch,
        compiler_params=pltpu.CompilerParams(
            needs_layout_passes=False, use_tc_tiling_on_sc=False),
    )(x_flat, params, emb_table)


def kernel(x, emb_table, fc_w, fc_b):
    x_flat = x.reshape(-1).astype(jnp.int32)
    params = jnp.concatenate(
        [fc_w.reshape(-1), jnp.broadcast_to(fc_b.reshape(-1), (16,))])
    out = _run(x_flat, params, emb_table)
    return out.reshape(BATCH, 1)
